# baseline (speedup 1.0000x reference)
import functools

import jax
import jax.numpy as jnp
from jax import lax
from jax.experimental import pallas as pl
from jax.experimental.pallas import tpu as pltpu
from jax.experimental.pallas import tpu_sc as plsc

B = 4096
L = 200
D = 64
DP = 128
V = 1000000
NUM_CLASSES = 10
NW = 32
RPW = B // NW
CHUNKS = (104, 96)
NLANE = D // 16
IPW = RPW * L
FW = 256
NBLK = V // FW
NT = (NBLK + NW - 1) // NW
VTAIL = NBLK * FW


def _sc_format(tt, tail128):
    mesh = plsc.VectorSubcoreMesh(core_axis_name="c", subcore_axis_name="s")

    @functools.partial(
        pl.kernel,
        mesh=mesh,
        out_type=jax.ShapeDtypeStruct((V, DP), jnp.float32),
        scratch_types=[
            pltpu.VMEM((2, D, FW), jnp.float32),
            pltpu.VMEM((2, FW, DP), jnp.float32),
            pltpu.SemaphoreType.DMA,
            pltpu.SemaphoreType.DMA,
            pltpu.SemaphoreType.DMA,
            pltpu.SemaphoreType.DMA,
        ],
        compiler_params=pltpu.CompilerParams(needs_layout_passes=False),
    )
    def k(tt_hbm, tail_hbm, t_hbm, in_v, out_v, is0, is1, os0, os1):
        cid = lax.axis_index("c")
        sid = lax.axis_index("s")
        wid = sid * 2 + cid
        isems = (is0, is1)
        osems = (os0, os1)
        lanes = lax.iota(jnp.int32, 16)
        n_t = (NBLK - wid + NW - 1) // NW
        s_list = [(lanes + kk) & 15 for kk in range(16)]
        ridx_list = [dc * 16 + lanes for dc in range(NLANE)]

        def in_copy(j, slot):
            return pltpu.make_async_copy(
                tt_hbm.at[:, pl.ds(j * FW, FW)], in_v.at[slot], isems[slot])

        def out_copy(j, slot):
            return pltpu.make_async_copy(
                out_v.at[slot], t_hbm.at[pl.ds(j * FW, FW)], osems[slot])

        def transpose(slot):
            in2d = in_v.at[slot]
            out2d = out_v.at[slot]

            @plsc.parallel_loop(0, FW // 16)
            def vblk(v0i):
                v0 = v0i * 16
                for dc in range(NLANE):
                    ridx = ridx_list[dc]
                    vvs = [v0 + s_list[kk] for kk in range(16)]
                    xs = [plsc.load_gather(in2d, [ridx, vvs[kk]])
                          for kk in range(16)]
                    for kk in range(16):
                        plsc.store_scatter(out2d, [vvs[kk], ridx], xs[kk])

        in_copy(wid, 0).start()

        def outer(g, carry):
            for b in range(2):
                t = g * 2 + b
                j = wid + NW * t

                @pl.when(j < NBLK)
                def _():
                    @pl.when(j + NW < NBLK)
                    def _():
                        in_copy(j + NW, 1 - b).start()

                    in_copy(j, b).wait()

                    @pl.when(t >= 2)
                    def _():
                        out_copy(j, b).wait()

                    transpose(b)
                    out_copy(j, b).start()
            return carry

        lax.fori_loop(0, (NT + 1) // 2, outer, 0)

        @pl.when(n_t % 2 == 1)
        def _():
            out_copy(0, 0).wait()

        @pl.when(n_t % 2 == 0)
        def _():
            out_copy(0, 1).wait()

        @pl.when(n_t >= 2)
        def _():
            @pl.when(n_t % 2 == 1)
            def _():
                out_copy(0, 1).wait()

            @pl.when(n_t % 2 == 0)
            def _():
                out_copy(0, 0).wait()

        @pl.when(wid == NW - 1)
        def _():
            pltpu.sync_copy(tail_hbm, t_hbm.at[pl.ds(VTAIL, 64)])

    return k(tt, tail128)


def _sc_pool(idx1, t128):
    mesh = plsc.VectorSubcoreMesh(core_axis_name="c", subcore_axis_name="s")

    @functools.partial(
        pl.kernel,
        mesh=mesh,
        out_type=jax.ShapeDtypeStruct((B, DP), jnp.float32),
        scratch_types=[
            pltpu.VMEM((IPW,), jnp.int32),
            pltpu.VMEM((2, L, DP), jnp.float32),
            pltpu.VMEM((RPW, DP), jnp.float32),
            pltpu.SemaphoreType.DMA,
            pltpu.SemaphoreType.DMA,
        ],
        compiler_params=pltpu.CompilerParams(needs_layout_passes=False),
    )
    def k(idx_hbm, table_hbm, cat_hbm, idx_v, rows_v, cat_buf, sem0, sem1):
        cid = lax.axis_index("c")
        sid = lax.axis_index("s")
        wid = sid * 2 + cid
        base = wid * RPW
        sems = (sem0, sem1)
        lanes = lax.iota(jnp.int32, 16)

        def issue(row, slot):
            off = 0
            for c in CHUNKS:
                pltpu.async_copy(
                    table_hbm.at[idx_v.at[pl.ds(row * L + off, c)]],
                    rows_v.at[slot, pl.ds(off, c)],
                    sems[slot],
                )
                off += c

        def wait_slot(slot):
            off = 0
            for c in CHUNKS:
                pltpu.make_async_copy(
                    table_hbm.at[idx_v.at[pl.ds(off, c)]],
                    rows_v.at[slot, pl.ds(off, c)],
                    sems[slot],
                ).wait()
                off += c

        def reduce_store(row, slot):
            rows2d = rows_v.at[slot]

            def body(i, carry):
                ivec = jnp.full((16,), i, jnp.int32)
                out = []
                for d in range(NLANE):
                    v = plsc.load_gather(rows2d, [ivec, d * 16 + lanes])
                    out.append(carry[2 * d] + v)
                    out.append(jnp.maximum(carry[2 * d + 1], v))
                return tuple(out)

            init = []
            for _ in range(NLANE):
                init.append(jnp.zeros((16,), jnp.float32))
                init.append(jnp.full((16,), -jnp.inf, jnp.float32))
            res = lax.fori_loop(0, L, body, tuple(init))
            rvec = jnp.full((16,), row, jnp.int32)
            for d in range(NLANE):
                plsc.store_scatter(cat_buf, [rvec, d * 16 + lanes],
                                   res[2 * d] * (1.0 / L))
                plsc.store_scatter(cat_buf, [rvec, D + d * 16 + lanes],
                                   res[2 * d + 1])

        pltpu.sync_copy(idx_hbm.at[pl.ds(base * L, IPW)], idx_v)
        issue(0, 0)

        def outer(g, carry):
            for b in range(2):
                row = g * 2 + b

                @pl.when(row + 1 < RPW)
                def _():
                    issue(row + 1, 1 - b)

                wait_slot(b)
                reduce_store(row, b)
            return carry

        lax.fori_loop(0, RPW // 2, outer, 0)

        pltpu.sync_copy(cat_buf, cat_hbm.at[pl.ds(base, RPW)])

    return k(idx1, t128)


def _head_body(cat_ref, w_ref, b_ref, out_ref):
    logits = (
        jnp.dot(cat_ref[...], w_ref[...], preferred_element_type=jnp.float32)
        + b_ref[...]
    )
    m = jnp.max(logits, axis=-1, keepdims=True)
    e = jnp.exp(logits - m)
    out_ref[...] = e / jnp.sum(e, axis=-1, keepdims=True)


def _tc_head(cat, fc_w, fc_b):
    b2 = fc_b.reshape(1, NUM_CLASSES)
    return pl.pallas_call(
        _head_body,
        out_shape=jax.ShapeDtypeStruct((B, NUM_CLASSES), jnp.float32),
    )(cat, fc_w, b2)


def kernel(inputs, table, fc_w, fc_b):
    idx1 = inputs.astype(jnp.int32).reshape(B * L)
    tt = table.T
    tail128 = jnp.pad(table[VTAIL:], ((0, 0), (0, DP - D)))
    t128 = _sc_format(tt, tail128)
    cat = _sc_pool(idx1, t128)
    return _tc_head(cat, fc_w, fc_b)


# speedup vs baseline: 1.0800x; 1.0800x over previous
import functools

import jax
import jax.numpy as jnp
from jax import lax
from jax.experimental import pallas as pl
from jax.experimental.pallas import tpu as pltpu
from jax.experimental.pallas import tpu_sc as plsc

B = 4096
L = 200
D = 64
DP = 128
V = 1000000
NUM_CLASSES = 10
NW = 32
RPW = B // NW
CHUNKS = (104, 96)
NLANE = D // 16
IPW = RPW * L
FW = 256
NBLK = V // FW
NT = (NBLK + NW - 1) // NW
VTAIL = NBLK * FW


def _sc_format(tt, tail128):
    mesh = plsc.VectorSubcoreMesh(core_axis_name="c", subcore_axis_name="s")

    @functools.partial(
        pl.kernel,
        mesh=mesh,
        out_type=jax.ShapeDtypeStruct((V, DP), jnp.float32),
        scratch_types=[
            pltpu.VMEM((2, D, FW), jnp.float32),
            pltpu.VMEM((2, FW, DP), jnp.float32),
            pltpu.SemaphoreType.DMA,
            pltpu.SemaphoreType.DMA,
            pltpu.SemaphoreType.DMA,
            pltpu.SemaphoreType.DMA,
        ],
        compiler_params=pltpu.CompilerParams(needs_layout_passes=False),
    )
    def k(tt_hbm, tail_hbm, t_hbm, in_v, out_v, is0, is1, os0, os1):
        cid = lax.axis_index("c")
        sid = lax.axis_index("s")
        wid = sid * 2 + cid
        isems = (is0, is1)
        osems = (os0, os1)
        lanes = lax.iota(jnp.int32, 16)
        n_t = (NBLK - wid + NW - 1) // NW
        s_list = [(lanes + kk) & 15 for kk in range(16)]
        ridx_list = [dc * 16 + lanes for dc in range(NLANE)]

        def in_copy(j, slot):
            return pltpu.make_async_copy(
                tt_hbm.at[:, pl.ds(j * FW, FW)], in_v.at[slot], isems[slot])

        def out_copy(j, slot):
            return pltpu.make_async_copy(
                out_v.at[slot], t_hbm.at[pl.ds(j * FW, FW)], osems[slot])

        def transpose(slot):
            in2d = in_v.at[slot]
            out2d = out_v.at[slot]

            def vblk(v0i, carry):
                v0 = v0i * 16
                for dc in range(NLANE):
                    ridx = ridx_list[dc]
                    vvs = [v0 + s_list[kk] for kk in range(16)]
                    xs = [plsc.load_gather(in2d, [ridx, vvs[kk]])
                          for kk in range(16)]
                    for kk in range(16):
                        plsc.store_scatter(out2d, [vvs[kk], ridx], xs[kk])
                return carry

            lax.fori_loop(0, FW // 16, vblk, 0)

        in_copy(wid, 0).start()

        def outer(g, carry):
            for b in range(2):
                t = g * 2 + b
                j = wid + NW * t

                @pl.when(j < NBLK)
                def _():
                    @pl.when(j + NW < NBLK)
                    def _():
                        in_copy(j + NW, 1 - b).start()

                    in_copy(j, b).wait()

                    @pl.when(t >= 2)
                    def _():
                        out_copy(j, b).wait()

                    transpose(b)
                    out_copy(j, b).start()
            return carry

        lax.fori_loop(0, (NT + 1) // 2, outer, 0)

        @pl.when(n_t % 2 == 1)
        def _():
            out_copy(0, 0).wait()

        @pl.when(n_t % 2 == 0)
        def _():
            out_copy(0, 1).wait()

        @pl.when(n_t >= 2)
        def _():
            @pl.when(n_t % 2 == 1)
            def _():
                out_copy(0, 1).wait()

            @pl.when(n_t % 2 == 0)
            def _():
                out_copy(0, 0).wait()

        @pl.when(wid == NW - 1)
        def _():
            pltpu.sync_copy(tail_hbm, t_hbm.at[pl.ds(VTAIL, 64)])

    return k(tt, tail128)


def _sc_pool(idx1, t128):
    mesh = plsc.VectorSubcoreMesh(core_axis_name="c", subcore_axis_name="s")

    @functools.partial(
        pl.kernel,
        mesh=mesh,
        out_type=jax.ShapeDtypeStruct((B, DP), jnp.float32),
        scratch_types=[
            pltpu.VMEM((IPW,), jnp.int32),
            pltpu.VMEM((3, L, DP), jnp.float32),
            pltpu.VMEM((RPW, DP), jnp.float32),
            pltpu.SemaphoreType.DMA,
            pltpu.SemaphoreType.DMA,
            pltpu.SemaphoreType.DMA,
        ],
        compiler_params=pltpu.CompilerParams(needs_layout_passes=False),
    )
    def k(idx_hbm, table_hbm, cat_hbm, idx_v, rows_v, cat_buf,
          sem0, sem1, sem2):
        cid = lax.axis_index("c")
        sid = lax.axis_index("s")
        wid = sid * 2 + cid
        base = wid * RPW
        sems = (sem0, sem1, sem2)
        lanes = lax.iota(jnp.int32, 16)

        def issue(row, slot):
            off = 0
            for c in CHUNKS:
                pltpu.async_copy(
                    table_hbm.at[idx_v.at[pl.ds(row * L + off, c)]],
                    rows_v.at[slot, pl.ds(off, c)],
                    sems[slot],
                )
                off += c

        def wait_slot(slot):
            off = 0
            for c in CHUNKS:
                pltpu.make_async_copy(
                    table_hbm.at[idx_v.at[pl.ds(off, c)]],
                    rows_v.at[slot, pl.ds(off, c)],
                    sems[slot],
                ).wait()
                off += c

        def reduce_store(row, slot):
            rows2d = rows_v.at[slot]

            def body(i, carry):
                ivec = jnp.full((16,), i, jnp.int32)
                out = []
                for d in range(NLANE):
                    v = plsc.load_gather(rows2d, [ivec, d * 16 + lanes])
                    out.append(carry[2 * d] + v)
                    out.append(jnp.maximum(carry[2 * d + 1], v))
                return tuple(out)

            init = []
            for _ in range(NLANE):
                init.append(jnp.zeros((16,), jnp.float32))
                init.append(jnp.full((16,), -jnp.inf, jnp.float32))
            res = lax.fori_loop(0, L, body, tuple(init))
            rvec = jnp.full((16,), row, jnp.int32)
            for d in range(NLANE):
                plsc.store_scatter(cat_buf, [rvec, d * 16 + lanes],
                                   res[2 * d] * (1.0 / L))
                plsc.store_scatter(cat_buf, [rvec, D + d * 16 + lanes],
                                   res[2 * d + 1])

        pltpu.sync_copy(idx_hbm.at[pl.ds(base * L, IPW)], idx_v)
        issue(0, 0)
        issue(1, 1)

        def outer(g, carry):
            for b in range(3):
                row = g * 3 + b

                @pl.when(row < RPW)
                def _():
                    @pl.when(row + 2 < RPW)
                    def _():
                        issue(row + 2, (b + 2) % 3)

                    wait_slot(b)
                    reduce_store(row, b)
            return carry

        lax.fori_loop(0, (RPW + 2) // 3, outer, 0)

        pltpu.sync_copy(cat_buf, cat_hbm.at[pl.ds(base, RPW)])

    return k(idx1, t128)


def _head_body(cat_ref, w_ref, b_ref, out_ref):
    logits = (
        jnp.dot(cat_ref[...], w_ref[...], preferred_element_type=jnp.float32)
        + b_ref[...]
    )
    m = jnp.max(logits, axis=-1, keepdims=True)
    e = jnp.exp(logits - m)
    out_ref[...] = e / jnp.sum(e, axis=-1, keepdims=True)


def _tc_head(cat, fc_w, fc_b):
    b2 = fc_b.reshape(1, NUM_CLASSES)
    return pl.pallas_call(
        _head_body,
        out_shape=jax.ShapeDtypeStruct((B, NUM_CLASSES), jnp.float32),
    )(cat, fc_w, b2)


def kernel(inputs, table, fc_w, fc_b):
    idx1 = inputs.astype(jnp.int32).reshape(B * L)
    tt = table.T
    tail128 = jnp.pad(table[VTAIL:], ((0, 0), (0, DP - D)))
    t128 = _sc_format(tt, tail128)
    cat = _sc_pool(idx1, t128)
    return _tc_head(cat, fc_w, fc_b)


# speedup vs baseline: 1.2901x; 1.1946x over previous
import functools

import jax
import jax.numpy as jnp
from jax import lax
from jax.experimental import pallas as pl
from jax.experimental.pallas import tpu as pltpu
from jax.experimental.pallas import tpu_sc as plsc

B = 4096
L = 200
D = 64
DP = 128
V = 1000000
NUM_CLASSES = 10
NW = 32
RPW = B // NW
CHUNKS = (104, 96)
NLANE = D // 16
IPW = RPW * L
FW = 256
NBLK = V // FW
NT = (NBLK + NW - 1) // NW
VTAIL = NBLK * FW
V2 = V // 2
TAIL_U = VTAIL // 2


def _sc_format(tt, tail128):
    mesh = plsc.VectorSubcoreMesh(core_axis_name="c", subcore_axis_name="s")

    @functools.partial(
        pl.kernel,
        mesh=mesh,
        out_type=jax.ShapeDtypeStruct((V2, DP), jnp.float32),
        scratch_types=[
            pltpu.VMEM((2, D, FW), jnp.float32),
            pltpu.VMEM((2, FW // 2, DP), jnp.float32),
            pltpu.SemaphoreType.DMA,
            pltpu.SemaphoreType.DMA,
            pltpu.SemaphoreType.DMA,
            pltpu.SemaphoreType.DMA,
        ],
        compiler_params=pltpu.CompilerParams(needs_layout_passes=False),
    )
    def k(tt_hbm, tail_hbm, t_hbm, in_v, out_v, is0, is1, os0, os1):
        cid = lax.axis_index("c")
        sid = lax.axis_index("s")
        wid = sid * 2 + cid
        isems = (is0, is1)
        osems = (os0, os1)
        lanes = lax.iota(jnp.int32, 16)
        n_t = (NBLK - wid + NW - 1) // NW
        s_list = [(lanes + kk) & 15 for kk in range(16)]
        sh_list = [s >> 1 for s in s_list]
        sp_list = [(s & 1) * D for s in s_list]
        ridx_list = [dc * 16 + lanes for dc in range(NLANE)]

        def in_copy(j, slot):
            return pltpu.make_async_copy(
                tt_hbm.at[:, pl.ds(j * FW, FW)], in_v.at[slot], isems[slot])

        def out_copy(j, slot):
            return pltpu.make_async_copy(
                out_v.at[slot], t_hbm.at[pl.ds(j * (FW // 2), FW // 2)],
                osems[slot])

        def transpose(slot):
            in2d = in_v.at[slot]
            out2d = out_v.at[slot]

            def vblk(v0i, carry):
                v0 = v0i * 16
                u0 = v0i * 8
                for dc in range(NLANE):
                    ridx = ridx_list[dc]
                    xs = [plsc.load_gather(in2d, [ridx, v0 + s_list[kk]])
                          for kk in range(16)]
                    for kk in range(16):
                        plsc.store_scatter(
                            out2d, [u0 + sh_list[kk], sp_list[kk] + ridx],
                            xs[kk])
                return carry

            lax.fori_loop(0, FW // 16, vblk, 0)

        in_copy(wid, 0).start()

        def outer(g, carry):
            for b in range(2):
                t = g * 2 + b
                j = wid + NW * t

                @pl.when(j < NBLK)
                def _():
                    @pl.when(j + NW < NBLK)
                    def _():
                        in_copy(j + NW, 1 - b).start()

                    in_copy(j, b).wait()

                    @pl.when(t >= 2)
                    def _():
                        out_copy(j, b).wait()

                    transpose(b)
                    out_copy(j, b).start()
            return carry

        lax.fori_loop(0, (NT + 1) // 2, outer, 0)

        @pl.when(n_t % 2 == 1)
        def _():
            out_copy(0, 0).wait()

        @pl.when(n_t % 2 == 0)
        def _():
            out_copy(0, 1).wait()

        @pl.when(n_t >= 2)
        def _():
            @pl.when(n_t % 2 == 1)
            def _():
                out_copy(0, 1).wait()

            @pl.when(n_t % 2 == 0)
            def _():
                out_copy(0, 0).wait()

        @pl.when(wid == NW - 1)
        def _():
            pltpu.sync_copy(tail_hbm, t_hbm.at[pl.ds(TAIL_U, 32)])

    return k(tt, tail128)


def _sc_pool(idx1, t128):
    mesh = plsc.VectorSubcoreMesh(core_axis_name="c", subcore_axis_name="s")

    @functools.partial(
        pl.kernel,
        mesh=mesh,
        out_type=jax.ShapeDtypeStruct((B, DP), jnp.float32),
        scratch_types=[
            pltpu.VMEM((IPW,), jnp.int32),
            pltpu.VMEM((IPW,), jnp.int32),
            pltpu.VMEM((3, L, DP), jnp.float32),
            pltpu.VMEM((8, DP), jnp.float32),
            pltpu.SemaphoreType.DMA,
            pltpu.SemaphoreType.DMA,
            pltpu.SemaphoreType.DMA,
        ],
        compiler_params=pltpu.CompilerParams(needs_layout_passes=False),
    )
    def k(idx_hbm, table_hbm, cat_hbm, par_v, idxh_v, rows_v, cat_buf,
          sem0, sem1, sem2):
        cid = lax.axis_index("c")
        sid = lax.axis_index("s")
        wid = sid * 2 + cid
        base = wid * RPW
        sems = (sem0, sem1, sem2)
        lanes = lax.iota(jnp.int32, 16)

        pltpu.sync_copy(idx_hbm.at[pl.ds(base * L, IPW)], par_v)

        def prep(kk, carry):
            raw = par_v[pl.ds(kk * 16, 16)]
            idxh_v[pl.ds(kk * 16, 16)] = raw >> 1
            par_v[pl.ds(kk * 16, 16)] = (raw & 1) << 6
            return carry

        lax.fori_loop(0, IPW // 16, prep, 0)

        def issue(row, slot):
            off = 0
            for c in CHUNKS:
                pltpu.async_copy(
                    table_hbm.at[idxh_v.at[pl.ds(row * L + off, c)]],
                    rows_v.at[slot, pl.ds(off, c)],
                    sems[slot],
                )
                off += c

        def wait_slot(slot):
            off = 0
            for c in CHUNKS:
                pltpu.make_async_copy(
                    table_hbm.at[idxh_v.at[pl.ds(off, c)]],
                    rows_v.at[slot, pl.ds(off, c)],
                    sems[slot],
                ).wait()
                off += c

        def reduce_store(row, slot):
            rows2d = rows_v.at[slot]

            def body(i, carry):
                par = plsc.load_gather(par_v, [jnp.full((16,), row * L + i,
                                                        jnp.int32)])
                ivec = jnp.full((16,), i, jnp.int32)
                out = []
                for d in range(NLANE):
                    v = plsc.load_gather(rows2d, [ivec, par + (d * 16) + lanes])
                    out.append(carry[2 * d] + v)
                    out.append(jnp.maximum(carry[2 * d + 1], v))
                return tuple(out)

            init = []
            for _ in range(NLANE):
                init.append(jnp.zeros((16,), jnp.float32))
                init.append(jnp.full((16,), -jnp.inf, jnp.float32))
            res = lax.fori_loop(0, L, body, tuple(init))
            rvec = jnp.full((16,), row & 7, jnp.int32)
            for d in range(NLANE):
                plsc.store_scatter(cat_buf, [rvec, d * 16 + lanes],
                                   res[2 * d] * (1.0 / L))
                plsc.store_scatter(cat_buf, [rvec, D + d * 16 + lanes],
                                   res[2 * d + 1])

            @pl.when((row & 7) == 7)
            def _():
                start = pl.multiple_of(base + row - 7, 8)
                pltpu.sync_copy(cat_buf, cat_hbm.at[pl.ds(start, 8)])

        issue(0, 0)
        issue(1, 1)

        def outer(g, carry):
            for b in range(3):
                row = g * 3 + b

                @pl.when(row < RPW)
                def _():
                    @pl.when(row + 2 < RPW)
                    def _():
                        issue(row + 2, (b + 2) % 3)

                    wait_slot(b)
                    reduce_store(row, b)
            return carry

        lax.fori_loop(0, (RPW + 2) // 3, outer, 0)

    return k(idx1, t128)


def _head_body(cat_ref, w_ref, b_ref, out_ref):
    logits = (
        jnp.dot(cat_ref[...], w_ref[...], preferred_element_type=jnp.float32)
        + b_ref[...]
    )
    m = jnp.max(logits, axis=-1, keepdims=True)
    e = jnp.exp(logits - m)
    out_ref[...] = e / jnp.sum(e, axis=-1, keepdims=True)


def _tc_head(cat, fc_w, fc_b):
    b2 = fc_b.reshape(1, NUM_CLASSES)
    return pl.pallas_call(
        _head_body,
        out_shape=jax.ShapeDtypeStruct((B, NUM_CLASSES), jnp.float32),
    )(cat, fc_w, b2)


def kernel(inputs, table, fc_w, fc_b):
    idx1 = inputs.astype(jnp.int32).reshape(B * L)
    tt = table.T
    tail2 = table[VTAIL:].reshape(32, DP)
    t2 = _sc_format(tt, tail2)
    cat = _sc_pool(idx1, t2)
    return _tc_head(cat, fc_w, fc_b)
